# Initial kernel scaffold; baseline (speedup 1.0000x reference)
import jax
import jax.numpy as jnp
from jax import lax
from jax.experimental import pallas as pl
from jax.experimental.pallas import tpu as pltpu
from jax.experimental.pallas import tpu_sc as plsc

N = 10000
E = 160000
NPAD = 10240
NG = 64
D = 256
HALF = 128
R = 256
NBLK = NPAD // R
NS = 16
K = 128
EPT = E // NS
CHUNKS = (EPT + K - 1) // K
EPT_PAD = CHUNKS * K
ROWS_PER_SUB = NPAD // NS
DUMMY_DST = NPAD - 2

_mesh = plsc.VectorSubcoreMesh(core_axis_name="c", subcore_axis_name="s")


def _hist_body(dst_hbm, ones_hbm, zeros_hbm, out_hbm, dst_v, ones_v,
               zeros_v, acc):
    c = lax.axis_index("c")
    s = lax.axis_index("s")
    pltpu.sync_copy(dst_hbm.at[s], dst_v)
    pltpu.sync_copy(ones_hbm, ones_v)
    pltpu.sync_copy(zeros_hbm, zeros_v)
    for k in range(ROWS_PER_SUB // K):
        pltpu.sync_copy(zeros_v, acc.at[pl.ds(s * ROWS_PER_SUB + k * K, K)])
    plsc.subcore_barrier()

    half = (CHUNKS + 1) // 2

    def body(j, carry):
        jj = c * half + j

        @pl.when(jj < CHUNKS)
        def _():
            pltpu.sync_copy(ones_v, acc.at[dst_v.at[jj]], add=True)

        return carry

    lax.fori_loop(0, half, body, 0)
    plsc.subcore_barrier()
    pltpu.sync_copy(acc.at[pl.ds(s * ROWS_PER_SUB, ROWS_PER_SUB)],
                    out_hbm.at[c].at[pl.ds(s * ROWS_PER_SUB, ROWS_PER_SUB)])


def _agg_body(table_hbm, src_hbm, dst_hbm, zeros_hbm, out_hbm,
              src_v, dst_v, gbuf, zeros_v, acc):
    c = lax.axis_index("c")
    s = lax.axis_index("s")
    pltpu.sync_copy(src_hbm.at[c].at[s], src_v)
    pltpu.sync_copy(dst_hbm.at[s], dst_v)
    pltpu.sync_copy(zeros_hbm, zeros_v)
    for k in range(ROWS_PER_SUB // K):
        pltpu.sync_copy(zeros_v, acc.at[pl.ds(s * ROWS_PER_SUB + k * K, K)])
    plsc.subcore_barrier()

    def body(j, carry):
        pltpu.sync_copy(table_hbm.at[src_v.at[j]], gbuf)
        pltpu.sync_copy(gbuf, acc.at[dst_v.at[j]], add=True)
        return carry

    lax.fori_loop(0, CHUNKS, body, 0)
    plsc.subcore_barrier()
    pltpu.sync_copy(acc.at[pl.ds(s * ROWS_PER_SUB, ROWS_PER_SUB)],
                    out_hbm.at[pl.ds(c * NPAD + s * ROWS_PER_SUB,
                                     ROWS_PER_SUB)])


def _sc_hist(dst_idx, ones16, zeros16):
    return pl.kernel(
        _hist_body,
        mesh=_mesh,
        out_type=jax.ShapeDtypeStruct((2, NPAD, 16), jnp.float32),
        scratch_types=[
            pltpu.VMEM((CHUNKS, K), jnp.int32),
            pltpu.VMEM((K, 16), jnp.float32),
            pltpu.VMEM((K, 16), jnp.float32),
            pltpu.VMEM_SHARED((NPAD, 16), jnp.float32),
        ],
    )(dst_idx, ones16, zeros16)


def _sc_agg(table_flat, src_idx, dst_idx, zeros128):
    return pl.kernel(
        _agg_body,
        mesh=_mesh,
        out_type=jax.ShapeDtypeStruct((2 * NPAD, HALF), jnp.float32),
        scratch_types=[
            pltpu.VMEM((CHUNKS, K), jnp.int32),
            pltpu.VMEM((CHUNKS, K), jnp.int32),
            pltpu.VMEM((K, HALF), jnp.float32),
            pltpu.VMEM((K, HALF), jnp.float32),
            pltpu.VMEM_SHARED((NPAD, HALF), jnp.float32),
        ],
    )(table_flat, src_idx, dst_idx, zeros128)


def _dinv_of(h0_ref, h1_ref):
    deg = h0_ref[:, 0] + h1_ref[:, 0] + 1.0
    return (1.0 / jnp.sqrt(deg))[:, None]


def _mm1_body(x_ref, w_ref, h0_ref, h1_ref, out_ref):
    dinv = _dinv_of(h0_ref, h1_ref)
    acc = jnp.dot(x_ref[...], w_ref[...], preferred_element_type=jnp.float32,
                  precision=lax.Precision.HIGHEST)
    out_ref[0] = acc * dinv


def _mm2_body(agg_ref, tab_ref, h0_ref, h1_ref, b1_ref, w_ref, out_ref):
    dinv = _dinv_of(h0_ref, h1_ref)
    m = agg_ref[...] + tab_ref[...]
    a = jnp.concatenate([m[0], m[1]], axis=1) * dinv + b1_ref[...]
    a = jnp.maximum(a, 0.0)
    acc = jnp.dot(a, w_ref[...], preferred_element_type=jnp.float32,
                  precision=lax.Precision.HIGHEST)
    out_ref[0] = acc * dinv


def _pool_body(agg_ref, tab_ref, h0_ref, h1_ref, b2_ref, batch_ref,
               fw1_ref, fb1_ref, fw2_ref, fb2_ref, out_ref, gmax):
    i = pl.program_id(0)

    @pl.when(i == 0)
    def _():
        gmax[...] = jnp.full((NG, D), -jnp.inf, jnp.float32)

    dinv = _dinv_of(h0_ref, h1_ref)
    m = agg_ref[...] + tab_ref[...]
    h = jnp.concatenate([m[0], m[1]], axis=1) * dinv + b2_ref[...]
    h = jnp.maximum(h, 0.0)
    bvec = batch_ref[0, 0]
    upd = []
    for g in range(NG):
        mask = (bvec == g)[:, None]
        upd.append(jnp.max(jnp.where(mask, h, -jnp.inf), axis=0,
                           keepdims=True))
    gmax[...] = jnp.maximum(gmax[...], jnp.concatenate(upd, axis=0))

    @pl.when(i == NBLK - 1)
    def _():
        g1 = jnp.dot(gmax[...], fw1_ref[...],
                     preferred_element_type=jnp.float32,
                     precision=lax.Precision.HIGHEST) + fb1_ref[...]
        g1 = jnp.maximum(g1, 0.0)
        out_ref[...] = jnp.dot(g1, fw2_ref[...],
                               preferred_element_type=jnp.float32,
                               precision=lax.Precision.HIGHEST) + fb2_ref[...]


def _mm1(x_pad, W1, h0, h1):
    return pl.pallas_call(
        _mm1_body,
        grid=(NBLK, 2),
        in_specs=[
            pl.BlockSpec((R, D), lambda i, c: (i, 0)),
            pl.BlockSpec((D, HALF), lambda i, c: (0, c)),
            pl.BlockSpec((R, 16), lambda i, c: (i, 0)),
            pl.BlockSpec((R, 16), lambda i, c: (i, 0)),
        ],
        out_specs=pl.BlockSpec((1, R, HALF), lambda i, c: (c, i, 0)),
        out_shape=jax.ShapeDtypeStruct((2, NPAD, HALF), jnp.float32),
    )(x_pad, W1, h0, h1)


def _mm2(agg1, tab1, h0, h1, b1r, W2):
    return pl.pallas_call(
        _mm2_body,
        grid=(NBLK, 2),
        in_specs=[
            pl.BlockSpec((2, R, HALF), lambda i, c: (0, i, 0)),
            pl.BlockSpec((2, R, HALF), lambda i, c: (0, i, 0)),
            pl.BlockSpec((R, 16), lambda i, c: (i, 0)),
            pl.BlockSpec((R, 16), lambda i, c: (i, 0)),
            pl.BlockSpec((1, D), lambda i, c: (0, 0)),
            pl.BlockSpec((D, HALF), lambda i, c: (0, c)),
        ],
        out_specs=pl.BlockSpec((1, R, HALF), lambda i, c: (c, i, 0)),
        out_shape=jax.ShapeDtypeStruct((2, NPAD, HALF), jnp.float32),
    )(agg1, tab1, h0, h1, b1r, W2)


def _pool(agg2, tab2, h0, h1, b2r, batch3, fcW1, fb1r, fw2p, fb2r):
    return pl.pallas_call(
        _pool_body,
        grid=(NBLK,),
        in_specs=[
            pl.BlockSpec((2, R, HALF), lambda i: (0, i, 0)),
            pl.BlockSpec((2, R, HALF), lambda i: (0, i, 0)),
            pl.BlockSpec((R, 16), lambda i: (i, 0)),
            pl.BlockSpec((R, 16), lambda i: (i, 0)),
            pl.BlockSpec((1, D), lambda i: (0, 0)),
            pl.BlockSpec((1, 1, R), lambda i: (i, 0, 0)),
            pl.BlockSpec((D, HALF), lambda i: (0, 0)),
            pl.BlockSpec((1, HALF), lambda i: (0, 0)),
            pl.BlockSpec((HALF, HALF), lambda i: (0, 0)),
            pl.BlockSpec((1, HALF), lambda i: (0, 0)),
        ],
        out_specs=pl.BlockSpec((NG, HALF), lambda i: (0, 0)),
        out_shape=jax.ShapeDtypeStruct((NG, HALF), jnp.float32),
        scratch_shapes=[pltpu.VMEM((NG, D), jnp.float32)],
    )(agg2, tab2, h0, h1, b2r, batch3, fcW1, fb1r, fw2p, fb2r)


def kernel(x, edge_index, edge_attr, batch, W1, b1, W2, b2,
           fcW1, fcb1, fcW2, fcb2):
    del edge_attr
    f32 = jnp.float32

    src = edge_index[0]
    dst = edge_index[1]
    pad_e = NS * EPT_PAD - E
    src_p = jnp.concatenate([src, jnp.zeros((pad_e,), jnp.int32)])
    dst_p = jnp.concatenate(
        [dst, jnp.full((pad_e,), DUMMY_DST, jnp.int32)])
    src_slab = src_p.reshape(NS, CHUNKS, K)
    src_idx = jnp.stack([src_slab, src_slab + NPAD])
    dst_idx = dst_p.reshape(NS, CHUNKS, K)

    x_pad = jnp.pad(x, ((0, NPAD - N), (0, 0)))
    batch_p = jnp.concatenate(
        [batch, jnp.full((NPAD - N,), -1, jnp.int32)]).reshape(NBLK, 1, R)

    ones16 = jnp.ones((K, 16), f32)
    zeros16 = jnp.zeros((K, 16), f32)
    zeros128 = jnp.zeros((K, HALF), f32)

    b1r = b1.reshape(1, D)
    b2r = b2.reshape(1, D)
    fb1r = fcb1.reshape(1, HALF)
    fw2p = jnp.pad(fcW2, ((0, 0), (0, HALF - 1)))
    fb2r = jnp.broadcast_to(fcb2.reshape(1, 1), (1, HALF))

    hist = _sc_hist(dst_idx, ones16, zeros16)
    h0, h1 = hist[0], hist[1]

    tab1 = _mm1(x_pad, W1, h0, h1)
    agg1 = _sc_agg(tab1.reshape(2 * NPAD, HALF), src_idx, dst_idx,
                   zeros128).reshape(2, NPAD, HALF)

    tab2 = _mm2(agg1, tab1, h0, h1, b1r, W2)
    agg2 = _sc_agg(tab2.reshape(2 * NPAD, HALF), src_idx, dst_idx,
                   zeros128).reshape(2, NPAD, HALF)

    out_full = _pool(agg2, tab2, h0, h1, b2r, batch_p,
                     fcW1, fb1r, fw2p, fb2r)
    return out_full[:, :1]


# speedup vs baseline: 4.4063x; 4.4063x over previous
import jax
import jax.numpy as jnp
from jax import lax
from jax.experimental import pallas as pl
from jax.experimental.pallas import tpu as pltpu
from jax.experimental.pallas import tpu_sc as plsc

N = 10000
E = 160000
NPAD = 10240
NG = 64
D = 256
HALF = 128
R = 256
NBLK = NPAD // R
NS = 16
K = 128
EPT = E // NS
CHUNKS = (EPT + K - 1) // K
EPT_PAD = CHUNKS * K
NH = NPAD // 2
ACC_ROWS = 6144
DUMMY_ROW = ACC_ROWS - 2
ZCH = ACC_ROWS // NS // K
OPS = NH // NS

_mesh = plsc.VectorSubcoreMesh(core_axis_name="c", subcore_axis_name="s")


def _hist_body(dst_hbm, ones_hbm, zeros_hbm, out_hbm, dst_v, ones_v,
               zeros_v, acc):
    c = lax.axis_index("c")
    s = lax.axis_index("s")
    pltpu.sync_copy(ones_hbm, ones_v)
    pltpu.sync_copy(zeros_hbm, zeros_v)
    halfc = (CHUNKS + 1) // 2
    for p in range(2):
        pltpu.sync_copy(dst_hbm.at[p].at[s], dst_v)
        for k in range(ZCH):
            pltpu.sync_copy(zeros_v,
                            acc.at[pl.ds(s * (ACC_ROWS // NS) + k * K, K)])
        plsc.subcore_barrier()

        def body(j, carry):
            jj = c * halfc + j

            @pl.when(jj < CHUNKS)
            def _():
                pltpu.sync_copy(ones_v, acc.at[dst_v.at[jj]], add=True)

            return carry

        lax.fori_loop(0, halfc, body, 0)
        plsc.subcore_barrier()
        pltpu.sync_copy(acc.at[pl.ds(s * OPS, OPS)],
                        out_hbm.at[c].at[pl.ds(p * NH + s * OPS, OPS)])
        plsc.subcore_barrier()


def _agg_body(table_hbm, src_hbm, dst_hbm, zeros_hbm, out_hbm,
              src_v, dst_v, gbuf, zeros_v, acc):
    c = lax.axis_index("c")
    s = lax.axis_index("s")
    pltpu.sync_copy(src_hbm.at[c].at[s], src_v)
    pltpu.sync_copy(zeros_hbm, zeros_v)
    for p in range(2):
        pltpu.sync_copy(dst_hbm.at[p].at[s], dst_v)
        for k in range(ZCH):
            pltpu.sync_copy(zeros_v,
                            acc.at[pl.ds(s * (ACC_ROWS // NS) + k * K, K)])
        plsc.subcore_barrier()

        def body(j, carry):
            pltpu.sync_copy(table_hbm.at[src_v.at[j]], gbuf)
            pltpu.sync_copy(gbuf, acc.at[dst_v.at[j]], add=True)
            return carry

        lax.fori_loop(0, CHUNKS, body, 0)
        plsc.subcore_barrier()
        pltpu.sync_copy(acc.at[pl.ds(s * OPS, OPS)],
                        out_hbm.at[c].at[pl.ds(p * NH + s * OPS, OPS)])
        plsc.subcore_barrier()


def _sc_hist(dst_idx, ones128, zeros128):
    return pl.kernel(
        _hist_body,
        mesh=_mesh,
        out_type=jax.ShapeDtypeStruct((2, NPAD, HALF), jnp.float32),
        scratch_types=[
            pltpu.VMEM((CHUNKS, K), jnp.int32),
            pltpu.VMEM((K, HALF), jnp.float32),
            pltpu.VMEM((K, HALF), jnp.float32),
            pltpu.VMEM_SHARED((ACC_ROWS, HALF), jnp.float32),
        ],
    )(dst_idx, ones128, zeros128)


def _sc_agg(table_flat, src_idx, dst_idx, zeros128):
    return pl.kernel(
        _agg_body,
        mesh=_mesh,
        out_type=jax.ShapeDtypeStruct((2, NPAD, HALF), jnp.float32),
        scratch_types=[
            pltpu.VMEM((CHUNKS, K), jnp.int32),
            pltpu.VMEM((CHUNKS, K), jnp.int32),
            pltpu.VMEM((K, HALF), jnp.float32),
            pltpu.VMEM((K, HALF), jnp.float32),
            pltpu.VMEM_SHARED((ACC_ROWS, HALF), jnp.float32),
        ],
    )(table_flat, src_idx, dst_idx, zeros128)


def _dinv_of(h0_ref, h1_ref):
    deg = h0_ref[:, 0:1] + h1_ref[:, 0:1] + 1.0
    return 1.0 / jnp.sqrt(deg)


def _mm1_body(x_ref, w_ref, h0_ref, h1_ref, out_ref):
    dinv = _dinv_of(h0_ref, h1_ref)
    acc = jnp.dot(x_ref[...], w_ref[...], preferred_element_type=jnp.float32,
                  precision=lax.Precision.HIGHEST)
    out_ref[0] = acc * dinv


def _assemble(agg_ref, tab_ref):
    m = agg_ref[...] + tab_ref[...]
    return jnp.concatenate([m[0], m[1]], axis=1)


def _mm2_body(agg_ref, tab_ref, h0_ref, h1_ref, b1_ref, w_ref, out_ref):
    dinv = _dinv_of(h0_ref, h1_ref)
    a = _assemble(agg_ref, tab_ref) * dinv + b1_ref[...]
    a = jnp.maximum(a, 0.0)
    acc = jnp.dot(a, w_ref[...], preferred_element_type=jnp.float32,
                  precision=lax.Precision.HIGHEST)
    out_ref[0] = acc * dinv


def _pool_body(agg_ref, tab_ref, h0_ref, h1_ref, b2_ref, batch_ref,
               fw1_ref, fb1_ref, fw2_ref, fb2_ref, out_ref, gmax):
    i = pl.program_id(0)

    @pl.when(i == 0)
    def _():
        gmax[...] = jnp.full((NG, D), -jnp.inf, jnp.float32)

    dinv = _dinv_of(h0_ref, h1_ref)
    h = _assemble(agg_ref, tab_ref) * dinv + b2_ref[...]
    h = jnp.maximum(h, 0.0)
    bcol = batch_ref[0]
    upd = []
    for g in range(NG):
        mask = bcol == g
        upd.append(jnp.max(jnp.where(mask, h, -jnp.inf), axis=0,
                           keepdims=True))
    gmax[...] = jnp.maximum(gmax[...], jnp.concatenate(upd, axis=0))

    @pl.when(i == NBLK - 1)
    def _():
        g1 = jnp.dot(gmax[...], fw1_ref[...],
                     preferred_element_type=jnp.float32,
                     precision=lax.Precision.HIGHEST) + fb1_ref[...]
        g1 = jnp.maximum(g1, 0.0)
        out_ref[...] = jnp.dot(g1, fw2_ref[...],
                               preferred_element_type=jnp.float32,
                               precision=lax.Precision.HIGHEST) + fb2_ref[...]


def _mm1(x_pad, W1, h0, h1):
    return pl.pallas_call(
        _mm1_body,
        grid=(NBLK, 2),
        in_specs=[
            pl.BlockSpec((R, D), lambda i, c: (i, 0)),
            pl.BlockSpec((D, HALF), lambda i, c: (0, c)),
            pl.BlockSpec((R, HALF), lambda i, c: (i, 0)),
            pl.BlockSpec((R, HALF), lambda i, c: (i, 0)),
        ],
        out_specs=pl.BlockSpec((1, R, HALF), lambda i, c: (c, i, 0)),
        out_shape=jax.ShapeDtypeStruct((2, NPAD, HALF), jnp.float32),
    )(x_pad, W1, h0, h1)


def _mm2(agg1, tab1, h0, h1, b1r, W2):
    return pl.pallas_call(
        _mm2_body,
        grid=(NBLK, 2),
        in_specs=[
            pl.BlockSpec((2, R, HALF), lambda i, c: (0, i, 0)),
            pl.BlockSpec((2, R, HALF), lambda i, c: (0, i, 0)),
            pl.BlockSpec((R, HALF), lambda i, c: (i, 0)),
            pl.BlockSpec((R, HALF), lambda i, c: (i, 0)),
            pl.BlockSpec((1, D), lambda i, c: (0, 0)),
            pl.BlockSpec((D, HALF), lambda i, c: (0, c)),
        ],
        out_specs=pl.BlockSpec((1, R, HALF), lambda i, c: (c, i, 0)),
        out_shape=jax.ShapeDtypeStruct((2, NPAD, HALF), jnp.float32),
    )(agg1, tab1, h0, h1, b1r, W2)


def _pool(agg2, tab2, h0, h1, b2r, batch3, fcW1, fb1r, fw2p, fb2r):
    return pl.pallas_call(
        _pool_body,
        grid=(NBLK,),
        in_specs=[
            pl.BlockSpec((2, R, HALF), lambda i: (0, i, 0)),
            pl.BlockSpec((2, R, HALF), lambda i: (0, i, 0)),
            pl.BlockSpec((R, HALF), lambda i: (i, 0)),
            pl.BlockSpec((R, HALF), lambda i: (i, 0)),
            pl.BlockSpec((1, D), lambda i: (0, 0)),
            pl.BlockSpec((1, R, 1), lambda i: (i, 0, 0)),
            pl.BlockSpec((D, HALF), lambda i: (0, 0)),
            pl.BlockSpec((1, HALF), lambda i: (0, 0)),
            pl.BlockSpec((HALF, HALF), lambda i: (0, 0)),
            pl.BlockSpec((1, HALF), lambda i: (0, 0)),
        ],
        out_specs=pl.BlockSpec((NG, HALF), lambda i: (0, 0)),
        out_shape=jax.ShapeDtypeStruct((NG, HALF), jnp.float32),
        scratch_shapes=[pltpu.VMEM((NG, D), jnp.float32)],
    )(agg2, tab2, h0, h1, b2r, batch3, fcW1, fb1r, fw2p, fb2r)


def kernel(x, edge_index, edge_attr, batch, W1, b1, W2, b2,
           fcW1, fcb1, fcW2, fcb2):
    del edge_attr
    f32 = jnp.float32

    src = edge_index[0]
    dst = edge_index[1]
    pad_e = NS * EPT_PAD - E
    src_p = jnp.concatenate([src, jnp.zeros((pad_e,), jnp.int32)])
    dst_p = jnp.concatenate([dst, jnp.full((pad_e,), -1, jnp.int32)])
    src_slab = src_p.reshape(NS, CHUNKS, K)
    src_idx = jnp.stack([src_slab, src_slab + NPAD])
    dlo = jnp.where((dst_p >= 0) & (dst_p < NH), dst_p, DUMMY_ROW)
    dhi = jnp.where(dst_p >= NH, dst_p - NH, DUMMY_ROW)
    dst_idx = jnp.stack([dlo.reshape(NS, CHUNKS, K),
                         dhi.reshape(NS, CHUNKS, K)])
    x_pad = jnp.pad(x, ((0, NPAD - N), (0, 0)))
    batch_p = jnp.concatenate(
        [batch, jnp.full((NPAD - N,), -1, jnp.int32)]).reshape(NBLK, R, 1)

    ones128 = jnp.ones((K, HALF), f32)
    zeros128 = jnp.zeros((K, HALF), f32)

    b1r = b1.reshape(1, D)
    b2r = b2.reshape(1, D)
    fb1r = fcb1.reshape(1, HALF)
    fw2p = jnp.pad(fcW2, ((0, 0), (0, HALF - 1)))
    fb2r = jnp.broadcast_to(fcb2.reshape(1, 1), (1, HALF))

    hist = _sc_hist(dst_idx, ones128, zeros128)
    h0, h1 = hist[0], hist[1]

    tab1 = _mm1(x_pad, W1, h0, h1)
    agg1 = _sc_agg(tab1.reshape(2 * NPAD, HALF), src_idx, dst_idx,
                   zeros128).reshape(2, NPAD, HALF)

    tab2 = _mm2(agg1, tab1, h0, h1, b1r, W2)
    agg2 = _sc_agg(tab2.reshape(2 * NPAD, HALF), src_idx, dst_idx,
                   zeros128).reshape(2, NPAD, HALF)

    out_full = _pool(agg2, tab2, h0, h1, b2r, batch_p,
                     fcW1, fb1r, fw2p, fb2r)
    return out_full[:, :1]
